# baseline (speedup 1.0000x reference)
import jax
import jax.numpy as jnp
from jax.experimental import pallas as pl
from jax.experimental.pallas import tpu as pltpu
import functools

B, N, M = 4, 4096, 4096
FEATURE_DIM = 64
SPATIAL_DIM = 3
WIDTH = 128
K = 16

MLP_BLK = 1024


def _gelu(x):
    return 0.5 * x * (1.0 + jax.lax.erf(x * 0.7071067811865476))


def _mlp_kernel(x_ref, w_ref, W1_ref, b1_ref, W2_ref, b2_ref, W3_ref, b3_ref,
                num_ref, den_ref):
    x = x_ref[...]
    h = _gelu(x @ W1_ref[...] + b1_ref[...])
    h = _gelu(h @ W2_ref[...] + b2_ref[...])
    v = h @ W3_ref[...] + b3_ref[...]
    w = w_ref[...]
    wv = v * w
    q = MLP_BLK // K
    num_ref[...] = jnp.sum(wv.reshape(q, K, WIDTH), axis=1)
    den_ref[...] = jnp.sum(w.reshape(q, K), axis=1, keepdims=True)


def kernel(source_coords, query_coords, source_features, W1, b1, W2, b2, W3, b3):
    q2 = jnp.sum(query_coords ** 2, axis=-1)[:, :, None]
    s2 = jnp.sum(source_coords ** 2, axis=-1)[:, None, :]
    cross = jnp.einsum("bmd,bnd->bmn", query_coords, source_coords)
    d2 = jnp.maximum(q2 + s2 - 2.0 * cross, 0.0)
    neg_vals, nn_idx = jax.lax.top_k(-d2, K)
    nn_dist = jnp.sqrt(jnp.maximum(-neg_vals, 0.0))

    src_c = jax.vmap(lambda s, idx: s[idx])(source_coords, nn_idx)
    src_f = jax.vmap(lambda s, idx: s[idx])(source_features, nn_idx)
    query = jnp.broadcast_to(query_coords[:, :, None, :], (B, M, K, SPATIAL_DIM))
    rel = query - src_c
    edge_input = jnp.concatenate(
        [query, src_c, rel, nn_dist[..., None], src_f], axis=-1)

    sigma = jnp.maximum(jnp.median(nn_dist[..., -1:]), 1e-6)
    w = jnp.exp(-nn_dist[..., None] / sigma)

    IN_CH = edge_input.shape[-1]
    x = edge_input.reshape(B * M * K, IN_CH)
    wflat = w.reshape(B * M * K, 1)

    rows = B * M * K
    grid = (rows // MLP_BLK,)
    num, den = pl.pallas_call(
        _mlp_kernel,
        grid=grid,
        in_specs=[
            pl.BlockSpec((MLP_BLK, IN_CH), lambda i: (i, 0)),
            pl.BlockSpec((MLP_BLK, 1), lambda i: (i, 0)),
            pl.BlockSpec((IN_CH, WIDTH), lambda i: (0, 0)),
            pl.BlockSpec((1, WIDTH), lambda i: (0, 0)),
            pl.BlockSpec((WIDTH, WIDTH), lambda i: (0, 0)),
            pl.BlockSpec((1, WIDTH), lambda i: (0, 0)),
            pl.BlockSpec((WIDTH, WIDTH), lambda i: (0, 0)),
            pl.BlockSpec((1, WIDTH), lambda i: (0, 0)),
        ],
        out_specs=[
            pl.BlockSpec((MLP_BLK // K, WIDTH), lambda i: (i, 0)),
            pl.BlockSpec((MLP_BLK // K, 1), lambda i: (i, 0)),
        ],
        out_shape=[
            jax.ShapeDtypeStruct((rows // K, WIDTH), jnp.float32),
            jax.ShapeDtypeStruct((rows // K, 1), jnp.float32),
        ],
    )(x, wflat, W1, b1.reshape(1, WIDTH), W2, b2.reshape(1, WIDTH),
      W3, b3.reshape(1, WIDTH))

    out = num / jnp.maximum(den, 1e-6)
    return out.reshape(B, M, WIDTH)


# speedup vs baseline: 2.3247x; 2.3247x over previous
import jax
import jax.numpy as jnp
from jax.experimental import pallas as pl
from jax.experimental.pallas import tpu as pltpu
import functools

B, N, M = 4, 4096, 4096
FEATURE_DIM = 64
SPATIAL_DIM = 3
WIDTH = 128
K = 16

QTILE = 128
NCHUNK = N // K
MLP_BLK = 1024

_PREC = jax.lax.Precision.HIGHEST


def _oddeven_merge_sort_pairs(n):
    pairs = []

    def merge(lo, cnt, r):
        step = r * 2
        if step < cnt:
            merge(lo, cnt, step)
            merge(lo + r, cnt, step)
            for i in range(lo + r, lo + cnt - r, step):
                pairs.append((i, i + r))
        else:
            pairs.append((lo, lo + r))

    def sort(lo, cnt):
        if cnt > 1:
            m = cnt // 2
            sort(lo, m)
            sort(lo + m, m)
            merge(lo, cnt, 1)

    sort(0, n)
    return pairs


_SORT16_PAIRS = _oddeven_merge_sort_pairs(K)


def _topk_kernel(s8_ref, qt8_ref, out_ref):
    s8 = s8_ref[...].reshape(N, 8)
    qt8 = qt8_ref[...].reshape(8, QTILE)
    scoord = s8[:, 0:3]
    s2col = s8[:, 3:4]
    qcoord = qt8[0:3, :]
    q2row = qt8[3:4, :]
    cross = jax.lax.dot_general(
        scoord, qcoord, (((1,), (0,)), ((), ())),
        preferred_element_type=jnp.float32,
        precision=jax.lax.Precision.DEFAULT)
    d2 = jnp.maximum((q2row + s2col) - 2.0 * cross, 0.0)

    bits = jax.lax.bitcast_convert_type(d2, jnp.int32)
    sidx = jax.lax.broadcasted_iota(jnp.int32, (N, QTILE), 0)
    keys = jax.lax.bitwise_or(
        jax.lax.bitwise_and(bits, jnp.int32(~0xFFF)), sidx)

    keys = keys.reshape(K, NCHUNK, QTILE)
    planes = [keys[p] for p in range(K)]

    for (i, j) in _SORT16_PAIRS:
        a, b = planes[i], planes[j]
        planes[i] = jnp.minimum(a, b)
        planes[j] = jnp.maximum(a, b)

    rows = NCHUNK
    while rows > 1:
        half = rows // 2
        merged = [
            jnp.minimum(planes[p][:half], planes[K - 1 - p][half:])
            for p in range(K)
        ]
        for j in (8, 4, 2, 1):
            for p in range(K):
                if (p % (2 * j)) < j:
                    a, b = merged[p], merged[p + j]
                    merged[p] = jnp.minimum(a, b)
                    merged[p + j] = jnp.maximum(a, b)
        planes = merged
        rows = half

    out_ref[...] = jnp.concatenate(planes, axis=0)[None, None]


def _gelu(x):
    return 0.5 * x * (1.0 + jax.lax.erf(x * 0.7071067811865476))


def _mlp_kernel(x_ref, w_ref, W1_ref, b1_ref, W2_ref, b2_ref, W3_ref, b3_ref,
                num_ref, den_ref):
    x = x_ref[...]
    h = _gelu(jnp.dot(x, W1_ref[...], precision=_PREC) + b1_ref[...])
    h = _gelu(jnp.dot(h, W2_ref[...], precision=_PREC) + b2_ref[...])
    v = jnp.dot(h, W3_ref[...], precision=_PREC) + b3_ref[...]
    w = w_ref[...]
    wv = v * w
    q = MLP_BLK // K
    num_ref[...] = jnp.sum(wv.reshape(q, K, WIDTH), axis=1)
    den_ref[...] = jnp.sum(w.reshape(q, K), axis=1, keepdims=True)


def _topk_search(source_coords, query_coords):
    s2 = jnp.sum(source_coords ** 2, axis=-1, keepdims=True)
    q2 = jnp.sum(query_coords ** 2, axis=-1, keepdims=True)
    s8 = jnp.concatenate(
        [source_coords, s2,
         jnp.zeros((B, N, 4), jnp.float32)], axis=-1)
    qt8 = jnp.concatenate(
        [query_coords, q2,
         jnp.zeros((B, M, 4), jnp.float32)], axis=-1).transpose(0, 2, 1)

    ntile = M // QTILE
    out = pl.pallas_call(
        _topk_kernel,
        grid=(B, ntile),
        in_specs=[
            pl.BlockSpec((1, N, 8), lambda b, t: (b, 0, 0)),
            pl.BlockSpec((1, 8, QTILE), lambda b, t: (b, 0, t)),
        ],
        out_specs=pl.BlockSpec((1, 1, K, QTILE), lambda b, t: (b, t, 0, 0)),
        out_shape=jax.ShapeDtypeStruct((B, ntile, K, QTILE), jnp.int32),
    )(s8, qt8)
    keys = jnp.transpose(out, (0, 1, 3, 2)).reshape(B, M, K)
    nn_idx = jax.lax.bitwise_and(keys, jnp.int32(0xFFF))
    d2q = jax.lax.bitcast_convert_type(
        jax.lax.bitwise_and(keys, jnp.int32(~0xFFF)), jnp.float32)
    return nn_idx, jnp.sqrt(d2q)


def kernel(source_coords, query_coords, source_features, W1, b1, W2, b2, W3, b3):
    nn_idx, nn_dist = _topk_search(source_coords, query_coords)

    src_c = jax.vmap(lambda s, idx: s[idx])(source_coords, nn_idx)
    src_f = jax.vmap(lambda s, idx: s[idx])(source_features, nn_idx)
    query = jnp.broadcast_to(query_coords[:, :, None, :], (B, M, K, SPATIAL_DIM))
    rel = query - src_c
    edge_input = jnp.concatenate(
        [query, src_c, rel, nn_dist[..., None], src_f], axis=-1)

    sigma = jnp.maximum(jnp.median(nn_dist[..., -1:]), 1e-6)
    w = jnp.exp(-nn_dist[..., None] / sigma)

    IN_CH = edge_input.shape[-1]
    x = edge_input.reshape(B * M * K, IN_CH)
    wflat = w.reshape(B * M * K, 1)

    rows = B * M * K
    grid = (rows // MLP_BLK,)
    num, den = pl.pallas_call(
        _mlp_kernel,
        grid=grid,
        in_specs=[
            pl.BlockSpec((MLP_BLK, IN_CH), lambda i: (i, 0)),
            pl.BlockSpec((MLP_BLK, 1), lambda i: (i, 0)),
            pl.BlockSpec((IN_CH, WIDTH), lambda i: (0, 0)),
            pl.BlockSpec((1, WIDTH), lambda i: (0, 0)),
            pl.BlockSpec((WIDTH, WIDTH), lambda i: (0, 0)),
            pl.BlockSpec((1, WIDTH), lambda i: (0, 0)),
            pl.BlockSpec((WIDTH, WIDTH), lambda i: (0, 0)),
            pl.BlockSpec((1, WIDTH), lambda i: (0, 0)),
        ],
        out_specs=[
            pl.BlockSpec((MLP_BLK // K, WIDTH), lambda i: (i, 0)),
            pl.BlockSpec((MLP_BLK // K, 1), lambda i: (i, 0)),
        ],
        out_shape=[
            jax.ShapeDtypeStruct((rows // K, WIDTH), jnp.float32),
            jax.ShapeDtypeStruct((rows // K, 1), jnp.float32),
        ],
    )(x, wflat, W1, b1.reshape(1, WIDTH), W2, b2.reshape(1, WIDTH),
      W3, b3.reshape(1, WIDTH))

    out = num / jnp.maximum(den, 1e-6)
    return out.reshape(B, M, WIDTH)


# speedup vs baseline: 15.5494x; 6.6887x over previous
import jax
import jax.numpy as jnp
from jax.experimental import pallas as pl
from jax.experimental.pallas import tpu as pltpu
from jax.experimental.pallas import tpu_sc as plsc

B, N, M = 4, 4096, 4096
FEATURE_DIM = 64
SPATIAL_DIM = 3
WIDTH = 128
K = 16

QTILE = 128
NCHUNK = N // K
MLP_BLK = 1024
TROW = 128
GWIN = 128

_PREC = jax.lax.Precision.HIGHEST


def _oddeven_merge_sort_pairs(n):
    pairs = []

    def merge(lo, cnt, r):
        step = r * 2
        if step < cnt:
            merge(lo, cnt, step)
            merge(lo + r, cnt, step)
            for i in range(lo + r, lo + cnt - r, step):
                pairs.append((i, i + r))
        else:
            pairs.append((lo, lo + r))

    def sort(lo, cnt):
        if cnt > 1:
            m = cnt // 2
            sort(lo, m)
            sort(lo + m, m)
            merge(lo, cnt, 1)

    sort(0, n)
    return pairs


_SORT16_PAIRS = _oddeven_merge_sort_pairs(K)


def _topk_kernel(s8_ref, qt8_ref, out_ref):
    s8 = s8_ref[...].reshape(N, 8)
    qt8 = qt8_ref[...].reshape(8, QTILE)
    scoord = s8[:, 0:3]
    s2col = s8[:, 3:4]
    qcoord = qt8[0:3, :]
    q2row = qt8[3:4, :]
    cross = jax.lax.dot_general(
        scoord, qcoord, (((1,), (0,)), ((), ())),
        preferred_element_type=jnp.float32,
        precision=jax.lax.Precision.DEFAULT)
    d2 = jnp.maximum((q2row + s2col) - 2.0 * cross, 0.0)

    bits = jax.lax.bitcast_convert_type(d2, jnp.int32)
    sidx = jax.lax.broadcasted_iota(jnp.int32, (N, QTILE), 0)
    keys = jax.lax.bitwise_or(
        jax.lax.bitwise_and(bits, jnp.int32(~0xFFF)), sidx)

    keys = keys.reshape(K, NCHUNK, QTILE)
    planes = [keys[p] for p in range(K)]

    for (i, j) in _SORT16_PAIRS:
        a, b = planes[i], planes[j]
        planes[i] = jnp.minimum(a, b)
        planes[j] = jnp.maximum(a, b)

    rows = NCHUNK
    while rows > 1:
        half = rows // 2
        merged = [
            jnp.minimum(planes[p][:half], planes[K - 1 - p][half:])
            for p in range(K)
        ]
        for j in (8, 4, 2, 1):
            for p in range(K):
                if (p % (2 * j)) < j:
                    a, b = merged[p], merged[p + j]
                    merged[p] = jnp.minimum(a, b)
                    merged[p + j] = jnp.maximum(a, b)
        planes = merged
        rows = half

    out_ref[...] = jnp.concatenate(planes, axis=0)[None, None]


def _topk_search(source_coords, query_coords):
    s2 = jnp.sum(source_coords ** 2, axis=-1, keepdims=True)
    q2 = jnp.sum(query_coords ** 2, axis=-1, keepdims=True)
    s8 = jnp.concatenate(
        [source_coords, s2,
         jnp.zeros((B, N, 4), jnp.float32)], axis=-1)
    qt8 = jnp.concatenate(
        [query_coords, q2,
         jnp.zeros((B, M, 4), jnp.float32)], axis=-1).transpose(0, 2, 1)

    ntile = M // QTILE
    out = pl.pallas_call(
        _topk_kernel,
        grid=(B, ntile),
        in_specs=[
            pl.BlockSpec((1, N, 8), lambda b, t: (b, 0, 0)),
            pl.BlockSpec((1, 8, QTILE), lambda b, t: (b, 0, t)),
        ],
        out_specs=pl.BlockSpec((1, 1, K, QTILE), lambda b, t: (b, t, 0, 0)),
        out_shape=jax.ShapeDtypeStruct((B, ntile, K, QTILE), jnp.int32),
    )(s8, qt8)
    keys = jnp.transpose(out, (0, 1, 3, 2)).reshape(B, M, K)
    nn_idx = jax.lax.bitwise_and(keys, jnp.int32(0xFFF))
    d2q = jax.lax.bitcast_convert_type(
        jax.lax.bitwise_and(keys, jnp.int32(~0xFFF)), jnp.float32)
    return nn_idx, jnp.sqrt(d2q)


def _sc_gather(table, flat_idx):
    num_idx = flat_idx.shape[0]
    idx2 = flat_idx.reshape(1, num_idx)
    mesh = plsc.VectorSubcoreMesh(core_axis_name="core",
                                  subcore_axis_name="subcore")

    @pl.kernel(out_type=jax.ShapeDtypeStruct((num_idx, TROW), jnp.float32),
               mesh=mesh)
    def gather_kernel(t_hbm, i_hbm, o_hbm):
        def body(i_vmem, o_vmem):
            pltpu.sync_copy(t_hbm.at[i_vmem.at[0]], o_vmem)

        pltpu.emit_pipeline(
            body,
            grid=(num_idx // GWIN,),
            in_specs=[pl.BlockSpec((1, GWIN), index_map=lambda i: (0, i))],
            out_specs=[pl.BlockSpec((GWIN, TROW),
                                    index_map=lambda i: (i, 0))],
            core_axis_name=("core", "subcore"),
            dimension_semantics=(pltpu.PARALLEL,),
        )(i_hbm, o_hbm)

    return gather_kernel(table, idx2)


def _gelu(x):
    return 0.5 * x * (1.0 + jax.lax.erf(x * 0.7071067811865476))


def _mlp_kernel(g_ref, aux_ref, Wf_ref, Wq_ref, Wc_ref, wd_ref, b1_ref,
                W2_ref, b2_ref, W3_ref, b3_ref, num_ref, den_ref):
    g = g_ref[...]
    aux = aux_ref[...]
    f = g[:, 0:FEATURE_DIM]
    c = g[:, FEATURE_DIM:FEATURE_DIM + 3]
    q = aux[:, 0:3]
    d = aux[:, 3:4]
    w = aux[:, 4:5]
    h = (jnp.dot(f, Wf_ref[...], precision=_PREC)
         + jnp.dot(q, Wq_ref[...], precision=_PREC)
         + jnp.dot(c, Wc_ref[...], precision=_PREC)
         + d * wd_ref[...]
         + b1_ref[...])
    h = _gelu(h)
    h = _gelu(jnp.dot(h, W2_ref[...], precision=_PREC) + b2_ref[...])
    v = jnp.dot(h, W3_ref[...], precision=_PREC) + b3_ref[...]
    wv = v * w
    nq = MLP_BLK // K
    num_ref[...] = jnp.sum(wv.reshape(nq, K, WIDTH), axis=1)
    den_ref[...] = jnp.sum(w.reshape(nq, K), axis=1, keepdims=True)


def kernel(source_coords, query_coords, source_features, W1, b1, W2, b2, W3, b3):
    nn_idx, nn_dist = _topk_search(source_coords, query_coords)

    table = jnp.concatenate(
        [source_features, source_coords,
         jnp.zeros((B, N, TROW - FEATURE_DIM - SPATIAL_DIM), jnp.float32)],
        axis=-1).reshape(B * N, TROW)
    flat_idx = (nn_idx + (jnp.arange(B, dtype=jnp.int32) * N)[:, None, None]
                ).reshape(B * M * K)
    g = _sc_gather(table, flat_idx)

    sigma = jnp.maximum(jnp.median(nn_dist[..., -1:]), 1e-6)
    w = jnp.exp(-nn_dist / sigma)

    qexp = jnp.broadcast_to(query_coords[:, :, None, :], (B, M, K, 3))
    aux = jnp.concatenate(
        [qexp, nn_dist[..., None], w[..., None],
         jnp.zeros((B, M, K, 3), jnp.float32)], axis=-1
    ).reshape(B * M * K, 8)

    Wq = W1[0:3] + W1[6:9]
    Wc = W1[3:6] - W1[6:9]
    wd = W1[9:10]
    Wf = W1[10:74]

    rows = B * M * K
    grid = (rows // MLP_BLK,)
    num, den = pl.pallas_call(
        _mlp_kernel,
        grid=grid,
        in_specs=[
            pl.BlockSpec((MLP_BLK, TROW), lambda i: (i, 0)),
            pl.BlockSpec((MLP_BLK, 8), lambda i: (i, 0)),
            pl.BlockSpec((FEATURE_DIM, WIDTH), lambda i: (0, 0)),
            pl.BlockSpec((3, WIDTH), lambda i: (0, 0)),
            pl.BlockSpec((3, WIDTH), lambda i: (0, 0)),
            pl.BlockSpec((1, WIDTH), lambda i: (0, 0)),
            pl.BlockSpec((1, WIDTH), lambda i: (0, 0)),
            pl.BlockSpec((WIDTH, WIDTH), lambda i: (0, 0)),
            pl.BlockSpec((1, WIDTH), lambda i: (0, 0)),
            pl.BlockSpec((WIDTH, WIDTH), lambda i: (0, 0)),
            pl.BlockSpec((1, WIDTH), lambda i: (0, 0)),
        ],
        out_specs=[
            pl.BlockSpec((MLP_BLK // K, WIDTH), lambda i: (i, 0)),
            pl.BlockSpec((MLP_BLK // K, 1), lambda i: (i, 0)),
        ],
        out_shape=[
            jax.ShapeDtypeStruct((rows // K, WIDTH), jnp.float32),
            jax.ShapeDtypeStruct((rows // K, 1), jnp.float32),
        ],
    )(g, aux, Wf, Wq, Wc, wd, b1.reshape(1, WIDTH),
      W2, b2.reshape(1, WIDTH), W3, b3.reshape(1, WIDTH))

    out = num / jnp.maximum(den, 1e-6)
    return out.reshape(B, M, WIDTH)


# speedup vs baseline: 17.3890x; 1.1183x over previous
import jax
import jax.numpy as jnp
from jax.experimental import pallas as pl
from jax.experimental.pallas import tpu as pltpu
from jax.experimental.pallas import tpu_sc as plsc

B, N, M = 4, 4096, 4096
FEATURE_DIM = 64
SPATIAL_DIM = 3
WIDTH = 128
K = 16

QTILE = 128
NCHUNK = N // K
MLP_BLK = 2048
TROW = 128
GWIN = 128

_PREC = jax.lax.Precision.HIGHEST
_PREC_MLP = jax.lax.Precision.HIGHEST


def _oddeven_merge_sort_pairs(n):
    pairs = []

    def merge(lo, cnt, r):
        step = r * 2
        if step < cnt:
            merge(lo, cnt, step)
            merge(lo + r, cnt, step)
            for i in range(lo + r, lo + cnt - r, step):
                pairs.append((i, i + r))
        else:
            pairs.append((lo, lo + r))

    def sort(lo, cnt):
        if cnt > 1:
            m = cnt // 2
            sort(lo, m)
            sort(lo + m, m)
            merge(lo, cnt, 1)

    sort(0, n)
    return pairs


_SORT16_PAIRS = _oddeven_merge_sort_pairs(K)


def _topk_kernel(s8_ref, qt8_ref, out_ref):
    s8 = s8_ref[...].reshape(N, 8)
    qt8 = qt8_ref[...].reshape(8, QTILE)
    scoord = s8[:, 0:3]
    s2col = s8[:, 3:4]
    qcoord = qt8[0:3, :]
    q2row = qt8[3:4, :]
    cross = jax.lax.dot_general(
        scoord, qcoord, (((1,), (0,)), ((), ())),
        preferred_element_type=jnp.float32,
        precision=jax.lax.Precision.DEFAULT)
    d2 = jnp.maximum((q2row + s2col) - 2.0 * cross, 0.0)

    bits = jax.lax.bitcast_convert_type(d2, jnp.int32)
    sidx = jax.lax.broadcasted_iota(jnp.int32, (N, QTILE), 0)
    keys = jax.lax.bitwise_or(
        jax.lax.bitwise_and(bits, jnp.int32(~0xFFF)), sidx)

    keys = keys.reshape(K, NCHUNK, QTILE)
    planes = [keys[p] for p in range(K)]

    for (i, j) in _SORT16_PAIRS:
        a, b = planes[i], planes[j]
        planes[i] = jnp.minimum(a, b)
        planes[j] = jnp.maximum(a, b)

    rows = NCHUNK
    while rows > 1:
        half = rows // 2
        merged = [
            jnp.minimum(planes[p][:half], planes[K - 1 - p][half:])
            for p in range(K)
        ]
        for j in (8, 4, 2, 1):
            for p in range(K):
                if (p % (2 * j)) < j:
                    a, b = merged[p], merged[p + j]
                    merged[p] = jnp.minimum(a, b)
                    merged[p + j] = jnp.maximum(a, b)
        planes = merged
        rows = half

    out_ref[...] = jnp.concatenate(planes, axis=0)[None, None]


def _topk_search(source_coords, query_coords):
    s2 = jnp.sum(source_coords ** 2, axis=-1, keepdims=True)
    q2 = jnp.sum(query_coords ** 2, axis=-1, keepdims=True)
    s8 = jnp.concatenate(
        [source_coords, s2,
         jnp.zeros((B, N, 4), jnp.float32)], axis=-1)
    qt8 = jnp.concatenate(
        [query_coords, q2,
         jnp.zeros((B, M, 4), jnp.float32)], axis=-1).transpose(0, 2, 1)

    ntile = M // QTILE
    out = pl.pallas_call(
        _topk_kernel,
        grid=(B, ntile),
        in_specs=[
            pl.BlockSpec((1, N, 8), lambda b, t: (b, 0, 0)),
            pl.BlockSpec((1, 8, QTILE), lambda b, t: (b, 0, t)),
        ],
        out_specs=pl.BlockSpec((1, 1, K, QTILE), lambda b, t: (b, t, 0, 0)),
        out_shape=jax.ShapeDtypeStruct((B, ntile, K, QTILE), jnp.int32),
    )(s8, qt8)
    keys = jnp.transpose(out, (0, 1, 3, 2)).reshape(B, M, K)
    nn_idx = jax.lax.bitwise_and(keys, jnp.int32(0xFFF))
    d2q = jax.lax.bitcast_convert_type(
        jax.lax.bitwise_and(keys, jnp.int32(~0xFFF)), jnp.float32)
    return nn_idx, jnp.sqrt(d2q)


def _sc_gather(table, flat_idx):
    num_idx = flat_idx.shape[0]
    idx2 = flat_idx.reshape(1, num_idx)
    mesh = plsc.VectorSubcoreMesh(core_axis_name="core",
                                  subcore_axis_name="subcore")

    @pl.kernel(out_type=jax.ShapeDtypeStruct((num_idx, TROW), jnp.float32),
               mesh=mesh)
    def gather_kernel(t_hbm, i_hbm, o_hbm):
        def body(i_vmem, o_vmem):
            pltpu.sync_copy(t_hbm.at[i_vmem.at[0]], o_vmem)

        pltpu.emit_pipeline(
            body,
            grid=(num_idx // GWIN,),
            in_specs=[pl.BlockSpec((1, GWIN), index_map=lambda i: (0, i))],
            out_specs=[pl.BlockSpec((GWIN, TROW),
                                    index_map=lambda i: (i, 0))],
            core_axis_name=("core", "subcore"),
            dimension_semantics=(pltpu.PARALLEL,),
        )(i_hbm, o_hbm)

    return gather_kernel(table, idx2)


def _gelu(x):
    return 0.5 * x * (1.0 + jax.lax.erf(x * 0.7071067811865476))


def _mlp_kernel(g_ref, aux_ref, Wg_ref, Waux_ref, b1_ref,
                W2_ref, b2_ref, W3_ref, b3_ref, num_ref, den_ref):
    g = g_ref[...]
    aux = aux_ref[...]
    w = aux[:, 4:5]
    h = (jnp.dot(g, Wg_ref[...], precision=_PREC_MLP)
         + jnp.dot(aux, Waux_ref[...], precision=_PREC_MLP)
         + b1_ref[...])
    h = _gelu(h)
    h = _gelu(jnp.dot(h, W2_ref[...], precision=_PREC_MLP) + b2_ref[...])
    v = jnp.dot(h, W3_ref[...], precision=_PREC_MLP) + b3_ref[...]
    wv = v * w
    nq = MLP_BLK // K
    num_ref[...] = jnp.sum(wv.reshape(nq, K, WIDTH), axis=1)
    den_ref[...] = jnp.sum(w.reshape(nq, K), axis=1, keepdims=True)


def kernel(source_coords, query_coords, source_features, W1, b1, W2, b2, W3, b3):
    nn_idx, nn_dist = _topk_search(source_coords, query_coords)

    table = jnp.concatenate(
        [source_features, source_coords,
         jnp.zeros((B, N, TROW - FEATURE_DIM - SPATIAL_DIM), jnp.float32)],
        axis=-1).reshape(B * N, TROW)
    flat_idx = (nn_idx + (jnp.arange(B, dtype=jnp.int32) * N)[:, None, None]
                ).reshape(B * M * K)
    g = _sc_gather(table, flat_idx)

    sigma = jnp.maximum(jnp.median(nn_dist[..., -1:]), 1e-6)
    w = jnp.exp(-nn_dist / sigma)

    qexp = jnp.broadcast_to(query_coords[:, :, None, :], (B, M, K, 3))
    aux = jnp.concatenate(
        [qexp, nn_dist[..., None], w[..., None],
         jnp.zeros((B, M, K, 3), jnp.float32)], axis=-1
    ).reshape(B * M * K, 8)

    Wg = jnp.concatenate(
        [W1[10:74], W1[3:6] - W1[6:9],
         jnp.zeros((TROW - FEATURE_DIM - 3, WIDTH), jnp.float32)], axis=0)
    Waux = jnp.concatenate(
        [W1[0:3] + W1[6:9], W1[9:10],
         jnp.zeros((4, WIDTH), jnp.float32)], axis=0)

    rows = B * M * K
    grid = (rows // MLP_BLK,)
    num, den = pl.pallas_call(
        _mlp_kernel,
        grid=grid,
        in_specs=[
            pl.BlockSpec((MLP_BLK, TROW), lambda i: (i, 0)),
            pl.BlockSpec((MLP_BLK, 8), lambda i: (i, 0)),
            pl.BlockSpec((TROW, WIDTH), lambda i: (0, 0)),
            pl.BlockSpec((8, WIDTH), lambda i: (0, 0)),
            pl.BlockSpec((1, WIDTH), lambda i: (0, 0)),
            pl.BlockSpec((WIDTH, WIDTH), lambda i: (0, 0)),
            pl.BlockSpec((1, WIDTH), lambda i: (0, 0)),
            pl.BlockSpec((WIDTH, WIDTH), lambda i: (0, 0)),
            pl.BlockSpec((1, WIDTH), lambda i: (0, 0)),
        ],
        out_specs=[
            pl.BlockSpec((MLP_BLK // K, WIDTH), lambda i: (i, 0)),
            pl.BlockSpec((MLP_BLK // K, 1), lambda i: (i, 0)),
        ],
        out_shape=[
            jax.ShapeDtypeStruct((rows // K, WIDTH), jnp.float32),
            jax.ShapeDtypeStruct((rows // K, 1), jnp.float32),
        ],
    )(g, aux, Wg, Waux, b1.reshape(1, WIDTH),
      W2, b2.reshape(1, WIDTH), W3, b3.reshape(1, WIDTH))

    out = num / jnp.maximum(den, 1e-6)
    return out.reshape(B, M, WIDTH)


# speedup vs baseline: 21.1625x; 1.2170x over previous
import jax
import jax.numpy as jnp
from jax.experimental import pallas as pl
from jax.experimental.pallas import tpu as pltpu
from jax.experimental.pallas import tpu_sc as plsc

B, N, M = 4, 4096, 4096
FEATURE_DIM = 64
SPATIAL_DIM = 3
WIDTH = 128
K = 16

QTILE = 128
NCHUNK = N // K
MLP_BLK = 4096
TROW = 128
GWIN = 128

_PREC = jax.lax.Precision.HIGHEST
_PREC_MLP = jax.lax.Precision.HIGHEST


def _oddeven_merge_sort_pairs(n):
    pairs = []

    def merge(lo, cnt, r):
        step = r * 2
        if step < cnt:
            merge(lo, cnt, step)
            merge(lo + r, cnt, step)
            for i in range(lo + r, lo + cnt - r, step):
                pairs.append((i, i + r))
        else:
            pairs.append((lo, lo + r))

    def sort(lo, cnt):
        if cnt > 1:
            m = cnt // 2
            sort(lo, m)
            sort(lo + m, m)
            merge(lo, cnt, 1)

    sort(0, n)
    return pairs


_SORT16_PAIRS = _oddeven_merge_sort_pairs(K)


def _topk_kernel(s8_ref, qt8_ref, out_ref):
    s8 = s8_ref[...].reshape(N, 8)
    qt8 = qt8_ref[...].reshape(8, QTILE)
    scoord = s8[:, 0:3]
    s2col = s8[:, 3:4]
    qcoord = qt8[0:3, :]
    q2row = qt8[3:4, :]
    cross = jax.lax.dot_general(
        scoord, qcoord, (((1,), (0,)), ((), ())),
        preferred_element_type=jnp.float32,
        precision=jax.lax.Precision.DEFAULT)
    d2 = jnp.maximum((q2row + s2col) - 2.0 * cross, 0.0)

    bits = jax.lax.bitcast_convert_type(d2, jnp.int32)
    sidx = jax.lax.broadcasted_iota(jnp.int32, (N, QTILE), 0)
    keys = jax.lax.bitwise_or(
        jax.lax.bitwise_and(bits, jnp.int32(~0xFFF)), sidx)
    keys = jax.lax.bitcast_convert_type(keys, jnp.float32)

    keys = keys.reshape(K, NCHUNK, QTILE)
    planes = [keys[p] for p in range(K)]

    for (i, j) in _SORT16_PAIRS:
        a, b = planes[i], planes[j]
        planes[i] = jnp.minimum(a, b)
        planes[j] = jnp.maximum(a, b)

    rows = NCHUNK
    while rows > 1:
        half = rows // 2
        merged = [
            jnp.minimum(planes[p][:half], planes[K - 1 - p][half:])
            for p in range(K)
        ]
        for j in (8, 4, 2, 1):
            for p in range(K):
                if (p % (2 * j)) < j:
                    a, b = merged[p], merged[p + j]
                    merged[p] = jnp.minimum(a, b)
                    merged[p + j] = jnp.maximum(a, b)
        planes = merged
        rows = half

    out_ref[...] = jax.lax.bitcast_convert_type(
        jnp.concatenate(planes, axis=0), jnp.int32)[None, None]


def _topk_search(source_coords, query_coords):
    s2 = jnp.sum(source_coords ** 2, axis=-1, keepdims=True)
    q2 = jnp.sum(query_coords ** 2, axis=-1, keepdims=True)
    s8 = jnp.concatenate(
        [source_coords, s2,
         jnp.zeros((B, N, 4), jnp.float32)], axis=-1)
    qt8 = jnp.concatenate(
        [query_coords, q2,
         jnp.zeros((B, M, 4), jnp.float32)], axis=-1).transpose(0, 2, 1)

    ntile = M // QTILE
    out = pl.pallas_call(
        _topk_kernel,
        grid=(B, ntile),
        in_specs=[
            pl.BlockSpec((1, N, 8), lambda b, t: (b, 0, 0)),
            pl.BlockSpec((1, 8, QTILE), lambda b, t: (b, 0, t)),
        ],
        out_specs=pl.BlockSpec((1, 1, K, QTILE), lambda b, t: (b, t, 0, 0)),
        out_shape=jax.ShapeDtypeStruct((B, ntile, K, QTILE), jnp.int32),
    )(s8, qt8)
    keys = jnp.transpose(out, (0, 1, 3, 2)).reshape(B, M, K)
    nn_idx = jax.lax.bitwise_and(keys, jnp.int32(0xFFF))
    d2q = jax.lax.bitcast_convert_type(
        jax.lax.bitwise_and(keys, jnp.int32(~0xFFF)), jnp.float32)
    return nn_idx, jnp.sqrt(d2q)


def _sc_gather(table, flat_idx):
    num_idx = flat_idx.shape[0]
    idx2 = flat_idx.reshape(1, num_idx)
    mesh = plsc.VectorSubcoreMesh(core_axis_name="core",
                                  subcore_axis_name="subcore")

    @pl.kernel(out_type=jax.ShapeDtypeStruct((num_idx, TROW), jnp.float32),
               mesh=mesh)
    def gather_kernel(t_hbm, i_hbm, o_hbm):
        def body(i_vmem, o_vmem):
            pltpu.sync_copy(t_hbm.at[i_vmem.at[0]], o_vmem)

        pltpu.emit_pipeline(
            body,
            grid=(num_idx // GWIN,),
            in_specs=[pl.BlockSpec((1, GWIN), index_map=lambda i: (0, i))],
            out_specs=[pl.BlockSpec((GWIN, TROW),
                                    index_map=lambda i: (i, 0))],
            core_axis_name=("core", "subcore"),
            dimension_semantics=(pltpu.PARALLEL,),
        )(i_hbm, o_hbm)

    return gather_kernel(table, idx2)


def _gelu(x):
    return 0.5 * x * (1.0 + jax.lax.erf(x * 0.7071067811865476))


def _x3dot(x, Wh_ref, Wl_ref):
    xh = x.astype(jnp.bfloat16)
    xl = (x - xh.astype(jnp.float32)).astype(jnp.bfloat16)
    acc = jax.lax.dot_general(
        xl, Wh_ref[...], (((1,), (0,)), ((), ())),
        preferred_element_type=jnp.float32)
    acc = acc + jax.lax.dot_general(
        xh, Wl_ref[...], (((1,), (0,)), ((), ())),
        preferred_element_type=jnp.float32)
    return acc + jax.lax.dot_general(
        xh, Wh_ref[...], (((1,), (0,)), ((), ())),
        preferred_element_type=jnp.float32)


def _mlp_kernel(g_ref, aux_ref, Wgh_ref, Wgl_ref, Waux_ref, b1_ref,
                W2h_ref, W2l_ref, b2_ref, W3h_ref, W3l_ref, b3_ref,
                num_ref, den_ref):
    g = g_ref[...]
    aux = aux_ref[...]
    w = aux[:, 4:5]
    h = (_x3dot(g, Wgh_ref, Wgl_ref)
         + jnp.dot(aux, Waux_ref[...], precision=_PREC)
         + b1_ref[...])
    h = _gelu(h)
    h = _gelu(_x3dot(h, W2h_ref, W2l_ref) + b2_ref[...])
    v = _x3dot(h, W3h_ref, W3l_ref) + b3_ref[...]
    wv = v * w
    nq = MLP_BLK // K
    num_ref[...] = jnp.sum(wv.reshape(nq, K, WIDTH), axis=1)
    den_ref[...] = jnp.sum(w.reshape(nq, K), axis=1, keepdims=True)


def kernel(source_coords, query_coords, source_features, W1, b1, W2, b2, W3, b3):
    nn_idx, nn_dist = _topk_search(source_coords, query_coords)

    table = jnp.concatenate(
        [source_features, source_coords,
         jnp.zeros((B, N, TROW - FEATURE_DIM - SPATIAL_DIM), jnp.float32)],
        axis=-1).reshape(B * N, TROW)
    flat_idx = (nn_idx + (jnp.arange(B, dtype=jnp.int32) * N)[:, None, None]
                ).reshape(B * M * K)
    g = _sc_gather(table, flat_idx)

    sigma = jnp.maximum(jnp.median(nn_dist[..., -1:]), 1e-6)
    w = jnp.exp(-nn_dist / sigma)

    qexp = jnp.broadcast_to(query_coords[:, :, None, :], (B, M, K, 3))
    aux = jnp.concatenate(
        [qexp, nn_dist[..., None], w[..., None],
         jnp.zeros((B, M, K, 3), jnp.float32)], axis=-1
    ).reshape(B * M * K, 8)

    Wg = jnp.concatenate(
        [W1[10:74], W1[3:6] - W1[6:9],
         jnp.zeros((TROW - FEATURE_DIM - 3, WIDTH), jnp.float32)], axis=0)
    Waux = jnp.concatenate(
        [W1[0:3] + W1[6:9], W1[9:10],
         jnp.zeros((4, WIDTH), jnp.float32)], axis=0)

    def _split(Wm):
        hi = Wm.astype(jnp.bfloat16)
        lo = (Wm - hi.astype(jnp.float32)).astype(jnp.bfloat16)
        return hi, lo

    Wgh, Wgl = _split(Wg)
    W2h, W2l = _split(W2)
    W3h, W3l = _split(W3)

    rows = B * M * K
    grid = (rows // MLP_BLK,)
    num, den = pl.pallas_call(
        _mlp_kernel,
        grid=grid,
        in_specs=[
            pl.BlockSpec((MLP_BLK, TROW), lambda i: (i, 0)),
            pl.BlockSpec((MLP_BLK, 8), lambda i: (i, 0)),
            pl.BlockSpec((TROW, WIDTH), lambda i: (0, 0)),
            pl.BlockSpec((TROW, WIDTH), lambda i: (0, 0)),
            pl.BlockSpec((8, WIDTH), lambda i: (0, 0)),
            pl.BlockSpec((1, WIDTH), lambda i: (0, 0)),
            pl.BlockSpec((WIDTH, WIDTH), lambda i: (0, 0)),
            pl.BlockSpec((WIDTH, WIDTH), lambda i: (0, 0)),
            pl.BlockSpec((1, WIDTH), lambda i: (0, 0)),
            pl.BlockSpec((WIDTH, WIDTH), lambda i: (0, 0)),
            pl.BlockSpec((WIDTH, WIDTH), lambda i: (0, 0)),
            pl.BlockSpec((1, WIDTH), lambda i: (0, 0)),
        ],
        out_specs=[
            pl.BlockSpec((MLP_BLK // K, WIDTH), lambda i: (i, 0)),
            pl.BlockSpec((MLP_BLK // K, 1), lambda i: (i, 0)),
        ],
        out_shape=[
            jax.ShapeDtypeStruct((rows // K, WIDTH), jnp.float32),
            jax.ShapeDtypeStruct((rows // K, 1), jnp.float32),
        ],
    )(g, aux, Wgh, Wgl, Waux, b1.reshape(1, WIDTH),
      W2h, W2l, b2.reshape(1, WIDTH), W3h, W3l, b3.reshape(1, WIDTH))

    out = num / jnp.maximum(den, 1e-6)
    return out.reshape(B, M, WIDTH)


# speedup vs baseline: 25.5799x; 1.2087x over previous
import jax
import jax.numpy as jnp
from jax.experimental import pallas as pl
from jax.experimental.pallas import tpu as pltpu
from jax.experimental.pallas import tpu_sc as plsc

B, N, M = 4, 4096, 4096
FEATURE_DIM = 64
SPATIAL_DIM = 3
WIDTH = 128
K = 16

QTILE = 128
NCHUNK = N // K
MLP_BLK = 4096
TROW = 128
GWIN = 128

_PREC = jax.lax.Precision.HIGHEST
_PREC_MLP = jax.lax.Precision.HIGHEST


def _oddeven_merge_sort_pairs(n):
    pairs = []

    def merge(lo, cnt, r):
        step = r * 2
        if step < cnt:
            merge(lo, cnt, step)
            merge(lo + r, cnt, step)
            for i in range(lo + r, lo + cnt - r, step):
                pairs.append((i, i + r))
        else:
            pairs.append((lo, lo + r))

    def sort(lo, cnt):
        if cnt > 1:
            m = cnt // 2
            sort(lo, m)
            sort(lo + m, m)
            merge(lo, cnt, 1)

    sort(0, n)
    return pairs


_SORT16_PAIRS = _oddeven_merge_sort_pairs(K)


def _topk_kernel(s8_ref, qt8_ref, out_ref):
    s8 = s8_ref[...].reshape(N, 8)
    qt8 = qt8_ref[...].reshape(8, QTILE)
    scoord = s8[:, 0:3]
    s2col = s8[:, 3:4]
    qcoord = qt8[0:3, :]
    q2row = qt8[3:4, :]
    cross = jax.lax.dot_general(
        scoord, qcoord, (((1,), (0,)), ((), ())),
        preferred_element_type=jnp.float32,
        precision=jax.lax.Precision.DEFAULT)
    d2 = jnp.maximum((q2row + s2col) - 2.0 * cross, 0.0)

    bits = jax.lax.bitcast_convert_type(d2, jnp.int32)
    sidx = jax.lax.broadcasted_iota(jnp.int32, (N, QTILE), 0)
    keys = jax.lax.bitwise_or(
        jax.lax.bitwise_and(bits, jnp.int32(~0xFFF)), sidx)

    keys = keys.reshape(K, NCHUNK, QTILE)
    planes = [keys[p] for p in range(K)]

    for (i, j) in _SORT16_PAIRS:
        a, b = planes[i], planes[j]
        planes[i] = jnp.minimum(a, b)
        planes[j] = jnp.maximum(a, b)

    rows = NCHUNK
    while rows > 1:
        half = rows // 2
        merged = [
            jnp.minimum(planes[p][:half], planes[K - 1 - p][half:])
            for p in range(K)
        ]
        for j in (8, 4, 2, 1):
            for p in range(K):
                if (p % (2 * j)) < j:
                    a, b = merged[p], merged[p + j]
                    merged[p] = jnp.minimum(a, b)
                    merged[p + j] = jnp.maximum(a, b)
        planes = merged
        rows = half

    out_ref[...] = jnp.concatenate(planes, axis=0)[None, None]


def _topk_search(source_coords, query_coords):
    s2 = jnp.sum(source_coords ** 2, axis=-1, keepdims=True)
    q2 = jnp.sum(query_coords ** 2, axis=-1, keepdims=True)
    s8 = jnp.concatenate(
        [source_coords, s2,
         jnp.zeros((B, N, 4), jnp.float32)], axis=-1)
    qt8 = jnp.concatenate(
        [query_coords, q2,
         jnp.zeros((B, M, 4), jnp.float32)], axis=-1).transpose(0, 2, 1)

    ntile = M // QTILE
    out = pl.pallas_call(
        _topk_kernel,
        grid=(B, ntile),
        in_specs=[
            pl.BlockSpec((1, N, 8), lambda b, t: (b, 0, 0)),
            pl.BlockSpec((1, 8, QTILE), lambda b, t: (b, 0, t)),
        ],
        out_specs=pl.BlockSpec((1, 1, K, QTILE), lambda b, t: (b, t, 0, 0)),
        out_shape=jax.ShapeDtypeStruct((B, ntile, K, QTILE), jnp.int32),
    )(s8, qt8)
    keys = jnp.transpose(out, (0, 1, 3, 2)).reshape(B, M, K)
    nn_idx = jax.lax.bitwise_and(keys, jnp.int32(0xFFF))
    d2q = jax.lax.bitcast_convert_type(
        jax.lax.bitwise_and(keys, jnp.int32(~0xFFF)), jnp.float32)
    return nn_idx, jnp.sqrt(d2q)


def _sc_gather(table, flat_idx):
    num_idx = flat_idx.shape[0]
    idx2 = flat_idx.reshape(1, num_idx)
    mesh = plsc.VectorSubcoreMesh(core_axis_name="core",
                                  subcore_axis_name="subcore")

    @pl.kernel(out_type=jax.ShapeDtypeStruct((num_idx, TROW), jnp.float32),
               mesh=mesh)
    def gather_kernel(t_hbm, i_hbm, o_hbm):
        def body(i_vmem, o_vmem):
            pltpu.sync_copy(t_hbm.at[i_vmem.at[0]], o_vmem)

        pltpu.emit_pipeline(
            body,
            grid=(num_idx // GWIN,),
            in_specs=[pl.BlockSpec((1, GWIN), index_map=lambda i: (0, i))],
            out_specs=[pl.BlockSpec((GWIN, TROW),
                                    index_map=lambda i: (i, 0))],
            core_axis_name=("core", "subcore"),
            dimension_semantics=(pltpu.PARALLEL,),
        )(i_hbm, o_hbm)

    return gather_kernel(table, idx2)


def _gelu(x):
    return 0.5 * x * (1.0 + jax.lax.erf(x * 0.7071067811865476))


def _x3dot(x, Wh_ref, Wl_ref):
    xh = x.astype(jnp.bfloat16)
    xl = (x - xh.astype(jnp.float32)).astype(jnp.bfloat16)
    acc = jax.lax.dot_general(
        xl, Wh_ref[...], (((1,), (0,)), ((), ())),
        preferred_element_type=jnp.float32)
    acc = acc + jax.lax.dot_general(
        xh, Wl_ref[...], (((1,), (0,)), ((), ())),
        preferred_element_type=jnp.float32)
    return acc + jax.lax.dot_general(
        xh, Wh_ref[...], (((1,), (0,)), ((), ())),
        preferred_element_type=jnp.float32)


def _mlp_kernel(g_ref, aux_ref, Wgh_ref, Wgl_ref, Waux_ref, b1_ref,
                W2h_ref, W2l_ref, b2_ref, W3h_ref, W3l_ref, b3_ref,
                num_ref, den_ref):
    g = g_ref[...]
    aux = aux_ref[...]
    w = aux[:, 4:5]
    h = (_x3dot(g, Wgh_ref, Wgl_ref)
         + jnp.dot(aux, Waux_ref[...], precision=_PREC)
         + b1_ref[...])
    h = _gelu(h)
    h = _gelu(_x3dot(h, W2h_ref, W2l_ref) + b2_ref[...])
    v = _x3dot(h, W3h_ref, W3l_ref) + b3_ref[...]
    wv = v * w
    nq = MLP_BLK // K
    num_ref[...] = jnp.sum(wv.reshape(nq, K, WIDTH), axis=1)
    den_ref[...] = jnp.sum(w.reshape(nq, K), axis=1, keepdims=True)


def kernel(source_coords, query_coords, source_features, W1, b1, W2, b2, W3, b3):
    nn_idx, nn_dist = _topk_search(source_coords, query_coords)

    table = jnp.concatenate(
        [source_features, source_coords,
         jnp.zeros((B, N, TROW - FEATURE_DIM - SPATIAL_DIM), jnp.float32)],
        axis=-1).reshape(B * N, TROW)
    flat_idx = (nn_idx + (jnp.arange(B, dtype=jnp.int32) * N)[:, None, None]
                ).reshape(B * M * K)
    g = _sc_gather(table, flat_idx)

    sigma = jnp.maximum(jnp.median(nn_dist[..., -1:]), 1e-6)
    w = jnp.exp(-nn_dist / sigma)

    qexp = jnp.broadcast_to(query_coords[:, :, None, :], (B, M, K, 3))
    aux = jnp.concatenate(
        [qexp, nn_dist[..., None], w[..., None],
         jnp.zeros((B, M, K, 3), jnp.float32)], axis=-1
    ).reshape(B * M * K, 8)

    Wg = jnp.concatenate(
        [W1[10:74], W1[3:6] - W1[6:9],
         jnp.zeros((TROW - FEATURE_DIM - 3, WIDTH), jnp.float32)], axis=0)
    Waux = jnp.concatenate(
        [W1[0:3] + W1[6:9], W1[9:10],
         jnp.zeros((4, WIDTH), jnp.float32)], axis=0)

    def _split(Wm):
        hi = Wm.astype(jnp.bfloat16)
        lo = (Wm - hi.astype(jnp.float32)).astype(jnp.bfloat16)
        return hi, lo

    Wgh, Wgl = _split(Wg)
    W2h, W2l = _split(W2)
    W3h, W3l = _split(W3)

    rows = B * M * K
    grid = (rows // MLP_BLK,)
    num, den = pl.pallas_call(
        _mlp_kernel,
        grid=grid,
        in_specs=[
            pl.BlockSpec((MLP_BLK, TROW), lambda i: (i, 0)),
            pl.BlockSpec((MLP_BLK, 8), lambda i: (i, 0)),
            pl.BlockSpec((TROW, WIDTH), lambda i: (0, 0)),
            pl.BlockSpec((TROW, WIDTH), lambda i: (0, 0)),
            pl.BlockSpec((8, WIDTH), lambda i: (0, 0)),
            pl.BlockSpec((1, WIDTH), lambda i: (0, 0)),
            pl.BlockSpec((WIDTH, WIDTH), lambda i: (0, 0)),
            pl.BlockSpec((WIDTH, WIDTH), lambda i: (0, 0)),
            pl.BlockSpec((1, WIDTH), lambda i: (0, 0)),
            pl.BlockSpec((WIDTH, WIDTH), lambda i: (0, 0)),
            pl.BlockSpec((WIDTH, WIDTH), lambda i: (0, 0)),
            pl.BlockSpec((1, WIDTH), lambda i: (0, 0)),
        ],
        out_specs=[
            pl.BlockSpec((MLP_BLK // K, WIDTH), lambda i: (i, 0)),
            pl.BlockSpec((MLP_BLK // K, 1), lambda i: (i, 0)),
        ],
        out_shape=[
            jax.ShapeDtypeStruct((rows // K, WIDTH), jnp.float32),
            jax.ShapeDtypeStruct((rows // K, 1), jnp.float32),
        ],
    )(g, aux, Wgh, Wgl, Waux, b1.reshape(1, WIDTH),
      W2h, W2l, b2.reshape(1, WIDTH), W3h, W3l, b3.reshape(1, WIDTH))

    out = num / jnp.maximum(den, 1e-6)
    return out.reshape(B, M, WIDTH)


# speedup vs baseline: 25.8688x; 1.0113x over previous
import jax
import jax.numpy as jnp
from jax.experimental import pallas as pl
from jax.experimental.pallas import tpu as pltpu
from jax.experimental.pallas import tpu_sc as plsc

B, N, M = 4, 4096, 4096
FEATURE_DIM = 64
SPATIAL_DIM = 3
WIDTH = 128
K = 16

QTILE = 128
NCHUNK = N // K
MLP_BLK = 4096
TROW = 128
GWIN = 128

_PREC = jax.lax.Precision.HIGHEST
_PREC_MLP = jax.lax.Precision.HIGHEST


def _oddeven_merge_sort_pairs(n):
    pairs = []

    def merge(lo, cnt, r):
        step = r * 2
        if step < cnt:
            merge(lo, cnt, step)
            merge(lo + r, cnt, step)
            for i in range(lo + r, lo + cnt - r, step):
                pairs.append((i, i + r))
        else:
            pairs.append((lo, lo + r))

    def sort(lo, cnt):
        if cnt > 1:
            m = cnt // 2
            sort(lo, m)
            sort(lo + m, m)
            merge(lo, cnt, 1)

    sort(0, n)
    return pairs


_SORT16_PAIRS = _oddeven_merge_sort_pairs(K)


def _topk_kernel(s8_ref, qt8_ref, out_ref):
    s8 = s8_ref[...].reshape(N, 8)
    qt8 = qt8_ref[...].reshape(8, QTILE)
    scoord = s8[:, 0:3]
    s2col = s8[:, 3:4]
    qcoord = qt8[0:3, :]
    q2row = qt8[3:4, :]
    cross = jax.lax.dot_general(
        scoord, qcoord, (((1,), (0,)), ((), ())),
        preferred_element_type=jnp.float32,
        precision=jax.lax.Precision.DEFAULT)
    d2 = jnp.maximum((q2row + s2col) - 2.0 * cross, 0.0)

    bits = jax.lax.bitcast_convert_type(d2, jnp.int32)
    sidx = jax.lax.broadcasted_iota(jnp.int32, (N, QTILE), 0)
    keys = jax.lax.bitwise_or(
        jax.lax.bitwise_and(bits, jnp.int32(~0xFFF)), sidx)

    keys = keys.reshape(K, NCHUNK, QTILE)
    planes = [keys[p] for p in range(K)]

    for (i, j) in _SORT16_PAIRS:
        a, b = planes[i], planes[j]
        planes[i] = jnp.minimum(a, b)
        planes[j] = jnp.maximum(a, b)

    rows = NCHUNK
    while rows > 1:
        half = rows // 2
        merged = [
            jnp.minimum(planes[p][:half], planes[K - 1 - p][half:])
            for p in range(K)
        ]
        for j in (8, 4, 2, 1):
            for p in range(K):
                if (p % (2 * j)) < j:
                    a, b = merged[p], merged[p + j]
                    merged[p] = jnp.minimum(a, b)
                    merged[p + j] = jnp.maximum(a, b)
        planes = merged
        rows = half

    out_ref[...] = jnp.concatenate(planes, axis=0)[None, None]


def _topk_half(s8, qt8_half):
    mh = qt8_half.shape[2]
    ntile = mh // QTILE
    out = pl.pallas_call(
        _topk_kernel,
        grid=(B, ntile),
        in_specs=[
            pl.BlockSpec((1, N, 8), lambda b, t: (b, 0, 0)),
            pl.BlockSpec((1, 8, QTILE), lambda b, t: (b, 0, t)),
        ],
        out_specs=pl.BlockSpec((1, 1, K, QTILE), lambda b, t: (b, t, 0, 0)),
        out_shape=jax.ShapeDtypeStruct((B, ntile, K, QTILE), jnp.int32),
    )(s8, qt8_half)
    keys = jnp.transpose(out, (0, 1, 3, 2)).reshape(B, mh, K)
    nn_idx = jax.lax.bitwise_and(keys, jnp.int32(0xFFF))
    d2q = jax.lax.bitcast_convert_type(
        jax.lax.bitwise_and(keys, jnp.int32(~0xFFF)), jnp.float32)
    return nn_idx, jnp.sqrt(d2q)


def _sc_gather(table, flat_idx):
    num_idx = flat_idx.shape[0]
    idx2 = flat_idx.reshape(1, num_idx)
    mesh = plsc.VectorSubcoreMesh(core_axis_name="core",
                                  subcore_axis_name="subcore")

    @pl.kernel(out_type=jax.ShapeDtypeStruct((num_idx, TROW), jnp.float32),
               mesh=mesh)
    def gather_kernel(t_hbm, i_hbm, o_hbm):
        def body(i_vmem, o_vmem):
            pltpu.sync_copy(t_hbm.at[i_vmem.at[0]], o_vmem)

        pltpu.emit_pipeline(
            body,
            grid=(num_idx // GWIN,),
            in_specs=[pl.BlockSpec((1, GWIN), index_map=lambda i: (0, i))],
            out_specs=[pl.BlockSpec((GWIN, TROW),
                                    index_map=lambda i: (i, 0))],
            core_axis_name=("core", "subcore"),
            dimension_semantics=(pltpu.PARALLEL,),
        )(i_hbm, o_hbm)

    return gather_kernel(table, idx2)


def _gelu(x):
    return 0.5 * x * (1.0 + jax.lax.erf(x * 0.7071067811865476))


def _x3dot(x, Wh_ref, Wl_ref):
    xh = x.astype(jnp.bfloat16)
    xl = (x - xh.astype(jnp.float32)).astype(jnp.bfloat16)
    acc = jax.lax.dot_general(
        xl, Wh_ref[...], (((1,), (0,)), ((), ())),
        preferred_element_type=jnp.float32)
    acc = acc + jax.lax.dot_general(
        xh, Wl_ref[...], (((1,), (0,)), ((), ())),
        preferred_element_type=jnp.float32)
    return acc + jax.lax.dot_general(
        xh, Wh_ref[...], (((1,), (0,)), ((), ())),
        preferred_element_type=jnp.float32)


def _mlp_kernel(g_ref, aux_ref, Wgh_ref, Wgl_ref, Waux_ref, b1_ref,
                W2h_ref, W2l_ref, b2_ref, W3h_ref, W3l_ref, b3_ref,
                num_ref, den_ref):
    g = g_ref[...]
    aux = aux_ref[...]
    w = aux[:, 4:5]
    h = (_x3dot(g, Wgh_ref, Wgl_ref)
         + jnp.dot(aux, Waux_ref[...], precision=_PREC)
         + b1_ref[...])
    h = _gelu(h)
    h = _gelu(_x3dot(h, W2h_ref, W2l_ref) + b2_ref[...])
    v = _x3dot(h, W3h_ref, W3l_ref) + b3_ref[...]
    wv = v * w
    nq = MLP_BLK // K
    num_ref[...] = jnp.sum(wv.reshape(nq, K, WIDTH), axis=1)
    den_ref[...] = jnp.sum(w.reshape(nq, K), axis=1, keepdims=True)


def _mlp_half(g, aux, Wgh, Wgl, Waux, b1, W2h, W2l, b2, W3h, W3l, b3):
    rows = g.shape[0]
    grid = (rows // MLP_BLK,)
    num, den = pl.pallas_call(
        _mlp_kernel,
        grid=grid,
        in_specs=[
            pl.BlockSpec((MLP_BLK, TROW), lambda i: (i, 0)),
            pl.BlockSpec((MLP_BLK, 8), lambda i: (i, 0)),
            pl.BlockSpec((TROW, WIDTH), lambda i: (0, 0)),
            pl.BlockSpec((TROW, WIDTH), lambda i: (0, 0)),
            pl.BlockSpec((8, WIDTH), lambda i: (0, 0)),
            pl.BlockSpec((1, WIDTH), lambda i: (0, 0)),
            pl.BlockSpec((WIDTH, WIDTH), lambda i: (0, 0)),
            pl.BlockSpec((WIDTH, WIDTH), lambda i: (0, 0)),
            pl.BlockSpec((1, WIDTH), lambda i: (0, 0)),
            pl.BlockSpec((WIDTH, WIDTH), lambda i: (0, 0)),
            pl.BlockSpec((WIDTH, WIDTH), lambda i: (0, 0)),
            pl.BlockSpec((1, WIDTH), lambda i: (0, 0)),
        ],
        out_specs=[
            pl.BlockSpec((MLP_BLK // K, WIDTH), lambda i: (i, 0)),
            pl.BlockSpec((MLP_BLK // K, 1), lambda i: (i, 0)),
        ],
        out_shape=[
            jax.ShapeDtypeStruct((rows // K, WIDTH), jnp.float32),
            jax.ShapeDtypeStruct((rows // K, 1), jnp.float32),
        ],
    )(g, aux, Wgh, Wgl, Waux, b1, W2h, W2l, b2, W3h, W3l, b3)
    return num / jnp.maximum(den, 1e-6)


def kernel(source_coords, query_coords, source_features, W1, b1, W2, b2, W3, b3):
    s2 = jnp.sum(source_coords ** 2, axis=-1, keepdims=True)
    q2 = jnp.sum(query_coords ** 2, axis=-1, keepdims=True)
    s8 = jnp.concatenate(
        [source_coords, s2,
         jnp.zeros((B, N, 4), jnp.float32)], axis=-1)
    qt8 = jnp.concatenate(
        [query_coords, q2,
         jnp.zeros((B, M, 4), jnp.float32)], axis=-1).transpose(0, 2, 1)

    MH = M // 2
    idx_d = [_topk_half(s8, qt8[:, :, :MH]), _topk_half(s8, qt8[:, :, MH:])]

    table = jnp.concatenate(
        [source_features, source_coords,
         jnp.zeros((B, N, TROW - FEATURE_DIM - SPATIAL_DIM), jnp.float32)],
        axis=-1).reshape(B * N, TROW)
    boff = (jnp.arange(B, dtype=jnp.int32) * N)[:, None, None]
    gs = [_sc_gather(table, (idx + boff).reshape(B * MH * K))
          for idx, _ in idx_d]

    sigma = jnp.maximum(
        jnp.median(jnp.concatenate(
            [d[..., -1:] for _, d in idx_d], axis=1)), 1e-6)

    qhalves = [query_coords[:, :MH], query_coords[:, MH:]]

    Wg = jnp.concatenate(
        [W1[10:74], W1[3:6] - W1[6:9],
         jnp.zeros((TROW - FEATURE_DIM - 3, WIDTH), jnp.float32)], axis=0)
    Waux = jnp.concatenate(
        [W1[0:3] + W1[6:9], W1[9:10],
         jnp.zeros((4, WIDTH), jnp.float32)], axis=0)

    def _split(Wm):
        hi = Wm.astype(jnp.bfloat16)
        lo = (Wm - hi.astype(jnp.float32)).astype(jnp.bfloat16)
        return hi, lo

    Wgh, Wgl = _split(Wg)
    W2h, W2l = _split(W2)
    W3h, W3l = _split(W3)
    b1r = b1.reshape(1, WIDTH)
    b2r = b2.reshape(1, WIDTH)
    b3r = b3.reshape(1, WIDTH)

    outs = []
    for (nn_idx, nn_dist), g, qh in zip(idx_d, gs, qhalves):
        w = jnp.exp(-nn_dist / sigma)
        qexp = jnp.broadcast_to(qh[:, :, None, :], (B, MH, K, 3))
        aux = jnp.concatenate(
            [qexp, nn_dist[..., None], w[..., None],
             jnp.zeros((B, MH, K, 3), jnp.float32)], axis=-1
        ).reshape(B * MH * K, 8)
        out_h = _mlp_half(g, aux, Wgh, Wgl, Waux, b1r,
                          W2h, W2l, b2r, W3h, W3l, b3r)
        outs.append(out_h.reshape(B, MH, WIDTH))

    return jnp.concatenate(outs, axis=1)
